# baseline (speedup 1.0000x reference)
import functools

import jax
import jax.numpy as jnp
from jax import lax
from jax.experimental import pallas as pl
from jax.experimental.pallas import tpu as pltpu
from jax.experimental.pallas import tpu_sc as plsc

N_NODES = 10000
N_EDGES = 320000
IN_DIM = 128
HID_DIM = 64
OUT_DIM = 32

NC = 2
NS = 16
NW = NC * NS
CHUNK = 128
E_PAD = 327680
EROWS = E_PAD // CHUNK
NCH = EROWS // NW
NBUF = 5
NROUND = NCH // NBUF
N_PAD = 10240
ROWS_SUB = N_PAD // NS
ZCH = 128
DEG_W = 16
BLK = 2000


def _zero_fill(buf, rows, width):
    @pl.loop(0, rows)
    def _(i):
        @pl.loop(0, width // 16)
        def _(j):
            buf[i, pl.ds(j * 16, 16)] = jnp.zeros((16,), jnp.float32)


def _make_agg(d):
    mesh = plsc.VectorSubcoreMesh(core_axis_name="c", subcore_axis_name="s")

    @functools.partial(
        pl.kernel,
        out_type=jax.ShapeDtypeStruct((NC, N_PAD, d), jnp.float32),
        mesh=mesh,
        compiler_params=pltpu.CompilerParams(use_tc_tiling_on_sc=False),
        scratch_types=[
            pltpu.VMEM((NCH, CHUNK), jnp.int32),
            pltpu.VMEM((NCH, CHUNK), jnp.int32),
            [pltpu.VMEM((CHUNK, d), jnp.float32)] * NBUF,
            pltpu.VMEM((ZCH, d), jnp.float32),
            pltpu.VMEM_SHARED((N_PAD, d), jnp.float32),
            [pltpu.SemaphoreType.DMA] * NBUF,
        ],
    )
    def agg(y_hbm, src_hbm, dst_hbm, out_hbm,
            src_idx, dst_idx, rows, zbuf, acc, gsem):
        cid = lax.axis_index("c")
        sid = lax.axis_index("s")
        w = cid * NS + sid
        base = sid * ROWS_SUB

        _zero_fill(zbuf, ZCH, d)

        @pl.loop(0, ROWS_SUB // ZCH)
        def _(k):
            pltpu.sync_copy(zbuf, acc.at[pl.ds(base + k * ZCH, ZCH)])

        pltpu.sync_copy(src_hbm.at[pl.ds(w * NCH, NCH)], src_idx)
        pltpu.sync_copy(dst_hbm.at[pl.ds(w * NCH, NCH)], dst_idx)
        plsc.subcore_barrier()

        def gstart(c, b):
            pltpu.async_copy(y_hbm.at[src_idx.at[c]], rows[b], gsem[b])

        def gwait(c, b):
            pltpu.make_async_copy(y_hbm.at[src_idx.at[c]], rows[b],
                                  gsem[b]).wait()

        def scat(c, b):
            pltpu.sync_copy(rows[b], acc.at[dst_idx.at[c]], add=True)

        for b in range(NBUF):
            gstart(b, b)

        @pl.loop(0, NROUND - 1)
        def _(k):
            c0 = k * NBUF
            for b in range(NBUF):
                gwait(c0 + b, b)
                scat(c0 + b, b)
                gstart(c0 + NBUF + b, b)

        c0 = (NROUND - 1) * NBUF
        for b in range(NBUF):
            gwait(c0 + b, b)
            scat(c0 + b, b)
        plsc.subcore_barrier()

        @pl.loop(0, ROWS_SUB // ZCH)
        def _(k):
            off = base + k * ZCH
            pltpu.sync_copy(acc.at[pl.ds(off, ZCH)],
                            out_hbm.at[cid, pl.ds(off, ZCH)])

    return agg


_agg_hid = _make_agg(HID_DIM)
_agg_out = _make_agg(OUT_DIM)


DEG_R = N_PAD // DEG_W
DEG_SUB = DEG_R // NS


def _make_deg():
    mesh = plsc.VectorSubcoreMesh(core_axis_name="c", subcore_axis_name="s")

    @functools.partial(
        pl.kernel,
        out_type=jax.ShapeDtypeStruct((NC, DEG_R, DEG_W), jnp.float32),
        mesh=mesh,
        compiler_params=pltpu.CompilerParams(use_tc_tiling_on_sc=False,
                                             needs_layout_passes=False),
        scratch_types=[
            pltpu.VMEM((NCH, CHUNK), jnp.int32),
            pltpu.VMEM((DEG_R, DEG_W), jnp.float32),
            pltpu.VMEM((DEG_R // ZCH + 1, ZCH), jnp.int32),
            pltpu.VMEM_SHARED((DEG_R, DEG_W), jnp.float32),
        ],
    )
    def deg(dst_hbm, out_hbm, dst_idx, hist, iden, acc):
        cid = lax.axis_index("c")
        sid = lax.axis_index("s")
        w = cid * NS + sid

        pltpu.sync_copy(dst_hbm.at[pl.ds(w * NCH, NCH)], dst_idx)
        _zero_fill(hist, DEG_R, DEG_W)
        for r in range(DEG_R // ZCH):
            for k in range(ZCH // 16):
                iden[r, pl.ds(k * 16, 16)] = (
                    lax.iota(jnp.int32, 16) + (r * ZCH + k * 16))
        pltpu.sync_copy(hist.at[pl.ds(0, DEG_SUB)],
                        acc.at[pl.ds(sid * DEG_SUB, DEG_SUB)])
        plsc.subcore_barrier()

        ones16 = jnp.full((16,), 1.0, jnp.float32)

        @pl.loop(0, NCH)
        def _(c):
            @pl.loop(0, CHUNK // 16)
            def _(j):
                d = dst_idx[c, pl.ds(j * 16, 16)]
                plsc.addupdate_scatter(
                    hist, [lax.shift_right_logical(d, 4), d & 15], ones16)

        for r in range(DEG_R // ZCH):
            pltpu.sync_copy(hist.at[pl.ds(r * ZCH, ZCH)],
                            acc.at[iden.at[r]], add=True)
        plsc.subcore_barrier()

        pltpu.sync_copy(acc.at[pl.ds(sid * DEG_SUB, DEG_SUB)],
                        out_hbm.at[cid, pl.ds(sid * DEG_SUB, DEG_SUB)])

    return deg


_deg = _make_deg()


def _mm1y_body(x_ref, w_ref, dc_ref, y_ref, dinv_ref):
    dinv = lax.rsqrt(dc_ref[...])
    dinv_ref[...] = dinv
    y_ref[...] = jnp.dot(x_ref[...], w_ref[...],
                         preferred_element_type=jnp.float32) * dinv


def _mm1y(x, W1, deg_col):
    return pl.pallas_call(
        _mm1y_body,
        grid=(N_NODES // BLK,),
        in_specs=[
            pl.BlockSpec((BLK, IN_DIM), lambda i: (i, 0)),
            pl.BlockSpec((IN_DIM, HID_DIM), lambda i: (0, 0)),
            pl.BlockSpec((BLK, 1), lambda i: (i, 0)),
        ],
        out_specs=(
            pl.BlockSpec((BLK, HID_DIM), lambda i: (i, 0)),
            pl.BlockSpec((BLK, 1), lambda i: (i, 0)),
        ),
        out_shape=(
            jax.ShapeDtypeStruct((N_NODES, HID_DIM), jnp.float32),
            jax.ShapeDtypeStruct((N_NODES, 1), jnp.float32),
        ),
    )(x, W1, deg_col)


def _mid_body(ap_ref, y1_ref, dinv_ref, b1_ref, w2_ref, y2_ref):
    dinv = dinv_ref[...]
    h = dinv * (ap_ref[0] + ap_ref[1] + y1_ref[...]) + b1_ref[...]
    h = jnp.maximum(h, 0.0)
    y2_ref[...] = jnp.dot(h, w2_ref[...],
                          preferred_element_type=jnp.float32) * dinv


def _mid(agg1, y1, dinv, b1, W2):
    return pl.pallas_call(
        _mid_body,
        grid=(N_NODES // BLK,),
        in_specs=[
            pl.BlockSpec((NC, BLK, HID_DIM), lambda i: (0, i, 0)),
            pl.BlockSpec((BLK, HID_DIM), lambda i: (i, 0)),
            pl.BlockSpec((BLK, 1), lambda i: (i, 0)),
            pl.BlockSpec((1, HID_DIM), lambda i: (0, 0)),
            pl.BlockSpec((HID_DIM, OUT_DIM), lambda i: (0, 0)),
        ],
        out_specs=pl.BlockSpec((BLK, OUT_DIM), lambda i: (i, 0)),
        out_shape=jax.ShapeDtypeStruct((N_NODES, OUT_DIM), jnp.float32),
    )(agg1, y1, dinv, b1.reshape(1, HID_DIM), W2)


def _final_body(ap_ref, y2_ref, dinv_ref, b2_ref, o_ref):
    o_ref[...] = (dinv_ref[...] * (ap_ref[0] + ap_ref[1] + y2_ref[...])
                  + b2_ref[...])


def _final(agg2, y2, dinv, b2):
    return pl.pallas_call(
        _final_body,
        grid=(N_NODES // BLK,),
        in_specs=[
            pl.BlockSpec((NC, BLK, OUT_DIM), lambda i: (0, i, 0)),
            pl.BlockSpec((BLK, OUT_DIM), lambda i: (i, 0)),
            pl.BlockSpec((BLK, 1), lambda i: (i, 0)),
            pl.BlockSpec((1, OUT_DIM), lambda i: (0, 0)),
        ],
        out_specs=pl.BlockSpec((BLK, OUT_DIM), lambda i: (i, 0)),
        out_shape=jax.ShapeDtypeStruct((N_NODES, OUT_DIM), jnp.float32),
    )(agg2, y2, dinv, b2.reshape(1, OUT_DIM))


def kernel(x, edge_index, W1, b1, W2, b2):
    npad = E_PAD - N_EDGES
    src = jnp.concatenate(
        [edge_index[0].astype(jnp.int32),
         jnp.zeros((npad,), jnp.int32)]).reshape(EROWS, CHUNK)
    dst = jnp.concatenate(
        [edge_index[1].astype(jnp.int32),
         N_NODES + (jnp.arange(npad, dtype=jnp.int32) % (N_PAD - N_NODES))]
    ).reshape(EROWS, CHUNK)

    deg_parts = _deg(dst)
    deg_col = (1.0 + (deg_parts[0] + deg_parts[1]).reshape(N_PAD)[:N_NODES]
               ).reshape(N_NODES, 1)
    y1, dinv = _mm1y(x, W1, deg_col)
    agg1 = _agg_hid(y1, src, dst)
    y2 = _mid(agg1, y1, dinv, b1, W2)
    agg2 = _agg_out(y2, src, dst)
    return _final(agg2, y2, dinv, b2)


# speedup vs baseline: 2.7007x; 2.7007x over previous
import functools

import jax
import jax.numpy as jnp
from jax import lax
from jax.experimental import pallas as pl
from jax.experimental.pallas import tpu as pltpu
from jax.experimental.pallas import tpu_sc as plsc

N_NODES = 10000
N_EDGES = 320000
IN_DIM = 128
HID_DIM = 64
OUT_DIM = 32

NC = 2
NS = 16
NW = NC * NS
CHUNK = 128
E_PAD = 327680
EROWS = E_PAD // CHUNK
NCH = EROWS // NW
NBUF = 5
NROUND = NCH // NBUF
N_PAD = 10240
ROWS_SUB = N_PAD // NS
ZCH = 128
DEG_W = 16
BLK = 2000


def _zero_fill(buf, rows, width):
    @pl.loop(0, rows)
    def _(i):
        @pl.loop(0, width // 16)
        def _(j):
            buf[i, pl.ds(j * 16, 16)] = jnp.zeros((16,), jnp.float32)


def _make_agg(d):
    mesh = plsc.VectorSubcoreMesh(core_axis_name="c", subcore_axis_name="s")

    @functools.partial(
        pl.kernel,
        out_type=jax.ShapeDtypeStruct((NC, N_PAD, d), jnp.float32),
        mesh=mesh,
        compiler_params=pltpu.CompilerParams(use_tc_tiling_on_sc=False),
        scratch_types=[
            pltpu.VMEM((NCH, CHUNK), jnp.int32),
            pltpu.VMEM((NCH, CHUNK), jnp.int32),
            [pltpu.VMEM((CHUNK, d), jnp.float32)] * NBUF,
            pltpu.VMEM((ZCH, d), jnp.float32),
            pltpu.VMEM_SHARED((N_PAD, d), jnp.float32),
            [pltpu.SemaphoreType.DMA] * NBUF,
        ],
    )
    def agg(y_hbm, src_hbm, dst_hbm, out_hbm,
            src_idx, dst_idx, rows, zbuf, acc, gsem):
        cid = lax.axis_index("c")
        sid = lax.axis_index("s")
        w = cid * NS + sid
        base = sid * ROWS_SUB

        _zero_fill(zbuf, ZCH, d)

        @pl.loop(0, ROWS_SUB // ZCH)
        def _(k):
            pltpu.sync_copy(zbuf, acc.at[pl.ds(base + k * ZCH, ZCH)])

        pltpu.sync_copy(src_hbm.at[pl.ds(w * NCH, NCH)], src_idx)
        pltpu.sync_copy(dst_hbm.at[pl.ds(w * NCH, NCH)], dst_idx)
        plsc.subcore_barrier()

        def gstart(c, b):
            pltpu.async_copy(y_hbm.at[src_idx.at[c]], rows[b], gsem[b])

        def gwait(c, b):
            pltpu.make_async_copy(y_hbm.at[src_idx.at[c]], rows[b],
                                  gsem[b]).wait()

        def scat(c, b):
            pltpu.sync_copy(rows[b], acc.at[dst_idx.at[c]], add=True)

        for b in range(NBUF):
            gstart(b, b)

        @pl.loop(0, NROUND - 1)
        def _(k):
            c0 = k * NBUF
            for b in range(NBUF):
                gwait(c0 + b, b)
                scat(c0 + b, b)
                gstart(c0 + NBUF + b, b)

        c0 = (NROUND - 1) * NBUF
        for b in range(NBUF):
            gwait(c0 + b, b)
            scat(c0 + b, b)
        plsc.subcore_barrier()

        @pl.loop(0, ROWS_SUB // ZCH)
        def _(k):
            off = base + k * ZCH
            pltpu.sync_copy(acc.at[pl.ds(off, ZCH)],
                            out_hbm.at[cid, pl.ds(off, ZCH)])

    return agg


_agg_hid = _make_agg(HID_DIM)
_agg_out = _make_agg(OUT_DIM)


DEG_R = N_PAD // DEG_W
DEG_SUB = DEG_R // NS


def _make_deg():
    mesh = plsc.VectorSubcoreMesh(core_axis_name="c", subcore_axis_name="s")

    @functools.partial(
        pl.kernel,
        out_type=jax.ShapeDtypeStruct((NC, DEG_R, DEG_W), jnp.float32),
        mesh=mesh,
        compiler_params=pltpu.CompilerParams(use_tc_tiling_on_sc=False,
                                             needs_layout_passes=False),
        scratch_types=[
            pltpu.VMEM((NCH, CHUNK), jnp.int32),
            pltpu.VMEM((DEG_R, DEG_W), jnp.float32),
            pltpu.VMEM((DEG_R // ZCH + 1, ZCH), jnp.int32),
            pltpu.VMEM_SHARED((DEG_R, DEG_W), jnp.float32),
        ],
    )
    def deg(dst_hbm, out_hbm, dst_idx, hist, iden, acc):
        cid = lax.axis_index("c")
        sid = lax.axis_index("s")
        w = cid * NS + sid

        pltpu.sync_copy(dst_hbm.at[pl.ds(w * NCH, NCH)], dst_idx)
        _zero_fill(hist, DEG_R, DEG_W)
        for r in range(DEG_R // ZCH):
            for k in range(ZCH // 16):
                iden[r, pl.ds(k * 16, 16)] = (
                    lax.iota(jnp.int32, 16) + (r * ZCH + k * 16))
        pltpu.sync_copy(hist.at[pl.ds(0, DEG_SUB)],
                        acc.at[pl.ds(sid * DEG_SUB, DEG_SUB)])
        plsc.subcore_barrier()

        ones16 = jnp.full((16,), 1.0, jnp.float32)

        @pl.loop(0, NCH)
        def _(c):
            @pl.loop(0, CHUNK // 16)
            def _(j):
                d = dst_idx[c, pl.ds(j * 16, 16)]
                plsc.addupdate_scatter(
                    hist, [lax.shift_right_logical(d, 4), d & 15], ones16)

        for r in range(DEG_R // ZCH):
            pltpu.sync_copy(hist.at[pl.ds(r * ZCH, ZCH)],
                            acc.at[iden.at[r]], add=True)
        plsc.subcore_barrier()

        pltpu.sync_copy(acc.at[pl.ds(sid * DEG_SUB, DEG_SUB)],
                        out_hbm.at[cid, pl.ds(sid * DEG_SUB, DEG_SUB)])

    return deg


_deg = _make_deg()


def _mm1y_body(x_ref, w_ref, dc_ref, y_ref, dinv_ref):
    dinv = lax.rsqrt(dc_ref[...])
    dinv_ref[...] = dinv
    y_ref[...] = jnp.dot(x_ref[...], w_ref[...],
                         preferred_element_type=jnp.float32) * dinv


def _mm1y(x, W1, deg_col):
    return pl.pallas_call(
        _mm1y_body,
        grid=(N_NODES // BLK,),
        in_specs=[
            pl.BlockSpec((BLK, IN_DIM), lambda i: (i, 0)),
            pl.BlockSpec((IN_DIM, HID_DIM), lambda i: (0, 0)),
            pl.BlockSpec((BLK, 1), lambda i: (i, 0)),
        ],
        out_specs=(
            pl.BlockSpec((BLK, HID_DIM), lambda i: (i, 0)),
            pl.BlockSpec((BLK, 1), lambda i: (i, 0)),
        ),
        out_shape=(
            jax.ShapeDtypeStruct((N_NODES, HID_DIM), jnp.float32),
            jax.ShapeDtypeStruct((N_NODES, 1), jnp.float32),
        ),
    )(x, W1, deg_col)


def _mid_body(ap_ref, y1_ref, dinv_ref, b1_ref, w2_ref, y2_ref):
    dinv = dinv_ref[...]
    h = dinv * (ap_ref[0] + ap_ref[1] + y1_ref[...]) + b1_ref[...]
    h = jnp.maximum(h, 0.0)
    y2_ref[...] = jnp.dot(h, w2_ref[...],
                          preferred_element_type=jnp.float32) * dinv


def _mid(agg1, y1, dinv, b1, W2):
    return pl.pallas_call(
        _mid_body,
        grid=(N_NODES // BLK,),
        in_specs=[
            pl.BlockSpec((NC, BLK, HID_DIM), lambda i: (0, i, 0)),
            pl.BlockSpec((BLK, HID_DIM), lambda i: (i, 0)),
            pl.BlockSpec((BLK, 1), lambda i: (i, 0)),
            pl.BlockSpec((1, HID_DIM), lambda i: (0, 0)),
            pl.BlockSpec((HID_DIM, OUT_DIM), lambda i: (0, 0)),
        ],
        out_specs=pl.BlockSpec((BLK, OUT_DIM), lambda i: (i, 0)),
        out_shape=jax.ShapeDtypeStruct((N_NODES, OUT_DIM), jnp.float32),
    )(agg1, y1, dinv, b1.reshape(1, HID_DIM), W2)


def _final_body(ap_ref, y2_ref, dinv_ref, b2_ref, o_ref):
    o_ref[...] = (dinv_ref[...] * (ap_ref[0] + ap_ref[1] + y2_ref[...])
                  + b2_ref[...])


def _final(agg2, y2, dinv, b2):
    return pl.pallas_call(
        _final_body,
        grid=(N_NODES // BLK,),
        in_specs=[
            pl.BlockSpec((NC, BLK, OUT_DIM), lambda i: (0, i, 0)),
            pl.BlockSpec((BLK, OUT_DIM), lambda i: (i, 0)),
            pl.BlockSpec((BLK, 1), lambda i: (i, 0)),
            pl.BlockSpec((1, OUT_DIM), lambda i: (0, 0)),
        ],
        out_specs=pl.BlockSpec((BLK, OUT_DIM), lambda i: (i, 0)),
        out_shape=jax.ShapeDtypeStruct((N_NODES, OUT_DIM), jnp.float32),
    )(agg2, y2, dinv, b2.reshape(1, OUT_DIM))


def kernel(x, edge_index, W1, b1, W2, b2):
    npad = E_PAD - N_EDGES
    spread = jnp.arange(npad, dtype=jnp.int32)
    src = jnp.concatenate(
        [edge_index[0].astype(jnp.int32),
         spread % N_NODES]).reshape(EROWS, CHUNK)
    dst = jnp.concatenate(
        [edge_index[1].astype(jnp.int32),
         N_NODES + spread % (N_PAD - N_NODES)]).reshape(EROWS, CHUNK)

    deg_parts = _deg(dst)
    deg_col = (1.0 + (deg_parts[0] + deg_parts[1]).reshape(N_PAD)[:N_NODES]
               ).reshape(N_NODES, 1)
    y1, dinv = _mm1y(x, W1, deg_col)
    agg1 = _agg_hid(y1, src, dst)
    y2 = _mid(agg1, y1, dinv, b1, W2)
    agg2 = _agg_out(y2, src, dst)
    return _final(agg2, y2, dinv, b2)


# speedup vs baseline: 2.7080x; 1.0027x over previous
import functools

import jax
import jax.numpy as jnp
from jax import lax
from jax.experimental import pallas as pl
from jax.experimental.pallas import tpu as pltpu
from jax.experimental.pallas import tpu_sc as plsc

N_NODES = 10000
N_EDGES = 320000
IN_DIM = 128
HID_DIM = 64
OUT_DIM = 32

NC = 2
NS = 16
NW = NC * NS
CHUNK = 128
E_PAD = 327680
EROWS = E_PAD // CHUNK
NCH = EROWS // NW
NBUF = 5
NROUND = NCH // NBUF
N_PAD = 10240
ROWS_SUB = N_PAD // NS
ZCH = 128
DEG_W = 16
BLK = 2000


def _zero_fill(buf, rows, width):
    @pl.loop(0, rows)
    def _(i):
        @pl.loop(0, width // 16)
        def _(j):
            buf[i, pl.ds(j * 16, 16)] = jnp.zeros((16,), jnp.float32)


def _make_agg(d):
    mesh = plsc.VectorSubcoreMesh(core_axis_name="c", subcore_axis_name="s")

    @functools.partial(
        pl.kernel,
        out_type=jax.ShapeDtypeStruct((NC, N_PAD, d), jnp.float32),
        mesh=mesh,
        compiler_params=pltpu.CompilerParams(use_tc_tiling_on_sc=False),
        scratch_types=[
            pltpu.VMEM((NCH, CHUNK), jnp.int32),
            pltpu.VMEM((NCH, CHUNK), jnp.int32),
            [pltpu.VMEM((CHUNK, d), jnp.float32)] * NBUF,
            pltpu.VMEM((ZCH, d), jnp.float32),
            pltpu.VMEM_SHARED((N_PAD, d), jnp.float32),
            [pltpu.SemaphoreType.DMA] * NBUF,
        ],
    )
    def agg(y_hbm, src_hbm, dst_hbm, out_hbm,
            src_idx, dst_idx, rows, zbuf, acc, gsem):
        cid = lax.axis_index("c")
        sid = lax.axis_index("s")
        w = cid * NS + sid
        base = sid * ROWS_SUB

        _zero_fill(zbuf, ZCH, d)

        @pl.loop(0, ROWS_SUB // ZCH)
        def _(k):
            pltpu.sync_copy(zbuf, acc.at[pl.ds(base + k * ZCH, ZCH)])

        pltpu.sync_copy(src_hbm.at[pl.ds(w * NCH, NCH)], src_idx)
        pltpu.sync_copy(dst_hbm.at[pl.ds(w * NCH, NCH)], dst_idx)
        plsc.subcore_barrier()

        def gstart(c, b):
            pltpu.async_copy(y_hbm.at[src_idx.at[c]], rows[b], gsem[b])

        def gwait(c, b):
            pltpu.make_async_copy(y_hbm.at[src_idx.at[c]], rows[b],
                                  gsem[b]).wait()

        def scat(c, b):
            pltpu.sync_copy(rows[b], acc.at[dst_idx.at[c]], add=True)

        for b in range(NBUF):
            gstart(b, b)

        @pl.loop(0, NROUND - 1)
        def _(k):
            c0 = k * NBUF
            for b in range(NBUF):
                gwait(c0 + b, b)
                scat(c0 + b, b)
                gstart(c0 + NBUF + b, b)

        c0 = (NROUND - 1) * NBUF
        for b in range(NBUF):
            gwait(c0 + b, b)
            scat(c0 + b, b)
        plsc.subcore_barrier()

        @pl.loop(0, ROWS_SUB // ZCH)
        def _(k):
            off = base + k * ZCH
            pltpu.sync_copy(acc.at[pl.ds(off, ZCH)],
                            out_hbm.at[cid, pl.ds(off, ZCH)])

    return agg


_agg_hid = _make_agg(HID_DIM)
_agg_out = _make_agg(OUT_DIM)


DEG_R = N_PAD // DEG_W
DEG_SUB = DEG_R // NS


def _make_deg():
    mesh = plsc.VectorSubcoreMesh(core_axis_name="c", subcore_axis_name="s")

    @functools.partial(
        pl.kernel,
        out_type=jax.ShapeDtypeStruct((NC, DEG_R, DEG_W), jnp.float32),
        mesh=mesh,
        compiler_params=pltpu.CompilerParams(use_tc_tiling_on_sc=False,
                                             needs_layout_passes=False),
        scratch_types=[
            pltpu.VMEM((NCH, CHUNK), jnp.int32),
            pltpu.VMEM((DEG_R, DEG_W), jnp.float32),
            pltpu.VMEM((DEG_R // ZCH + 1, ZCH), jnp.int32),
            pltpu.VMEM_SHARED((DEG_R, DEG_W), jnp.float32),
        ],
    )
    def deg(dst_hbm, out_hbm, dst_idx, hist, iden, acc):
        cid = lax.axis_index("c")
        sid = lax.axis_index("s")
        w = cid * NS + sid

        pltpu.sync_copy(dst_hbm.at[pl.ds(w * NCH, NCH)], dst_idx)
        _zero_fill(hist, DEG_R, DEG_W)
        for r in range(DEG_R // ZCH):
            for k in range(ZCH // 16):
                iden[r, pl.ds(k * 16, 16)] = (
                    lax.iota(jnp.int32, 16) + (r * ZCH + k * 16))
        pltpu.sync_copy(hist.at[pl.ds(0, DEG_SUB)],
                        acc.at[pl.ds(sid * DEG_SUB, DEG_SUB)])
        plsc.subcore_barrier()

        ones16 = jnp.full((16,), 1.0, jnp.float32)

        @pl.loop(0, NCH)
        def _(c):
            @pl.loop(0, CHUNK // 16)
            def _(j):
                d = dst_idx[c, pl.ds(j * 16, 16)]
                plsc.addupdate_scatter(
                    hist, [lax.shift_right_logical(d, 4), d & 15], ones16)

        for r in range(DEG_R // ZCH):
            pltpu.sync_copy(hist.at[pl.ds(r * ZCH, ZCH)],
                            acc.at[iden.at[r]], add=True)
        plsc.subcore_barrier()

        pltpu.sync_copy(acc.at[pl.ds(sid * DEG_SUB, DEG_SUB)],
                        out_hbm.at[cid, pl.ds(sid * DEG_SUB, DEG_SUB)])

    return deg


_deg = _make_deg()


def _mm1y_body(x_ref, w_ref, dc_ref, y_ref, dinv_ref):
    dinv = lax.rsqrt(dc_ref[...])
    dinv_ref[...] = dinv
    y_ref[...] = jnp.dot(x_ref[...], w_ref[...],
                         preferred_element_type=jnp.float32) * dinv


def _mm1y(x, W1, deg_col):
    return pl.pallas_call(
        _mm1y_body,
        grid=(N_NODES // BLK,),
        in_specs=[
            pl.BlockSpec((BLK, IN_DIM), lambda i: (i, 0)),
            pl.BlockSpec((IN_DIM, HID_DIM), lambda i: (0, 0)),
            pl.BlockSpec((BLK, 1), lambda i: (i, 0)),
        ],
        out_specs=(
            pl.BlockSpec((BLK, HID_DIM), lambda i: (i, 0)),
            pl.BlockSpec((BLK, 1), lambda i: (i, 0)),
        ),
        out_shape=(
            jax.ShapeDtypeStruct((N_NODES, HID_DIM), jnp.float32),
            jax.ShapeDtypeStruct((N_NODES, 1), jnp.float32),
        ),
    )(x, W1, deg_col)


def _mid_body(ap_ref, y1_ref, dinv_ref, b1_ref, w2_ref, y2_ref):
    dinv = dinv_ref[...]
    h = dinv * (ap_ref[0] + ap_ref[1] + y1_ref[...]) + b1_ref[...]
    h = jnp.maximum(h, 0.0)
    y2_ref[...] = jnp.dot(h, w2_ref[...],
                          preferred_element_type=jnp.float32) * dinv


def _mid(agg1, y1, dinv, b1, W2):
    return pl.pallas_call(
        _mid_body,
        grid=(N_NODES // BLK,),
        in_specs=[
            pl.BlockSpec((NC, BLK, HID_DIM), lambda i: (0, i, 0)),
            pl.BlockSpec((BLK, HID_DIM), lambda i: (i, 0)),
            pl.BlockSpec((BLK, 1), lambda i: (i, 0)),
            pl.BlockSpec((1, HID_DIM), lambda i: (0, 0)),
            pl.BlockSpec((HID_DIM, OUT_DIM), lambda i: (0, 0)),
        ],
        out_specs=pl.BlockSpec((BLK, OUT_DIM), lambda i: (i, 0)),
        out_shape=jax.ShapeDtypeStruct((N_NODES, OUT_DIM), jnp.float32),
    )(agg1, y1, dinv, b1.reshape(1, HID_DIM), W2)


def _final_body(ap_ref, y2_ref, dinv_ref, b2_ref, o_ref):
    o_ref[...] = (dinv_ref[...] * (ap_ref[0] + ap_ref[1] + y2_ref[...])
                  + b2_ref[...])


def _final(agg2, y2, dinv, b2):
    return pl.pallas_call(
        _final_body,
        grid=(N_NODES // BLK,),
        in_specs=[
            pl.BlockSpec((NC, BLK, OUT_DIM), lambda i: (0, i, 0)),
            pl.BlockSpec((BLK, OUT_DIM), lambda i: (i, 0)),
            pl.BlockSpec((BLK, 1), lambda i: (i, 0)),
            pl.BlockSpec((1, OUT_DIM), lambda i: (0, 0)),
        ],
        out_specs=pl.BlockSpec((BLK, OUT_DIM), lambda i: (i, 0)),
        out_shape=jax.ShapeDtypeStruct((N_NODES, OUT_DIM), jnp.float32),
    )(agg2, y2, dinv, b2.reshape(1, OUT_DIM))


def kernel(x, edge_index, W1, b1, W2, b2):
    npad = E_PAD - N_EDGES
    spread = jnp.arange(npad, dtype=jnp.int32)
    dst = jnp.concatenate(
        [edge_index[1].astype(jnp.int32),
         N_NODES + spread % (N_PAD - N_NODES)]).reshape(EROWS, CHUNK)
    dst = lax.optimization_barrier(dst)
    anchor = dst[0, 0] * 0
    src = jnp.concatenate(
        [edge_index[0].astype(jnp.int32),
         (spread + anchor) % N_NODES]).reshape(EROWS, CHUNK)

    deg_parts = _deg(dst)
    deg_col = (1.0 + (deg_parts[0] + deg_parts[1]).reshape(N_PAD)[:N_NODES]
               ).reshape(N_NODES, 1)
    y1, dinv = _mm1y(x, W1, deg_col)
    agg1 = _agg_hid(y1, src, dst)
    y2 = _mid(agg1, y1, dinv, b1, W2)
    agg2 = _agg_out(y2, src, dst)
    return _final(agg2, y2, dinv, b2)
